# Initial kernel scaffold; baseline (speedup 1.0000x reference)
import jax
import jax.numpy as jnp
from jax.experimental import pallas as pl


def kernel(x_disease, x_gene, src_disease, dst_gene, W_l_dg, b_l_dg, W_r_dg, W_l_gd, b_l_gd, W_r_gd, W_mu, b_mu, W_ls, b_ls):
    raise NotImplementedError("write your pallas kernel here")


# speedup vs baseline: 1.0938x; 1.0938x over previous
import functools

import jax
import jax.numpy as jnp
from jax.experimental import pallas as pl


N_D, N_G, E, D = 10000, 50000, 320000, 128


def _mm_body(x_ref, w_ref, y_ref):
    y_ref[...] = jnp.dot(x_ref[...], w_ref[...],
                         preferred_element_type=jnp.float32)


def _pre_matmul(x_d, W_l):
    R = 1000
    return pl.pallas_call(
        _mm_body,
        grid=(N_D // R,),
        in_specs=[
            pl.BlockSpec((R, D), lambda i: (i, 0)),
            pl.BlockSpec((D, D), lambda i: (0, 0)),
        ],
        out_specs=pl.BlockSpec((R, D), lambda i: (i, 0)),
        out_shape=jax.ShapeDtypeStruct((N_D, D), jnp.float32),
    )(x_d, W_l)


def _epi_body(agg_ref, cnt_ref, xg_ref, eps_ref, wr_ref, bl_ref,
              wmu_ref, bmu_ref, wls_ref, bls_ref, z_ref):
    cnt = jnp.maximum(cnt_ref[...], 1.0)
    h = (agg_ref[...] / cnt + bl_ref[...]
         + jnp.dot(xg_ref[...], wr_ref[...],
                   preferred_element_type=jnp.float32))
    mu = jnp.dot(h, wmu_ref[...], preferred_element_type=jnp.float32) + bmu_ref[...]
    ls = jnp.dot(h, wls_ref[...], preferred_element_type=jnp.float32) + bls_ref[...]
    z_ref[...] = mu + eps_ref[...] * jnp.exp(ls)


def _epilogue(agg, cnt, x_g, eps, W_r, b_l, W_mu, b_mu, W_ls, b_ls):
    R = 1000
    mat = lambda: pl.BlockSpec((R, D), lambda i: (i, 0))
    wgt = lambda: pl.BlockSpec((D, D), lambda i: (0, 0))
    vec = lambda: pl.BlockSpec((1, D), lambda i: (0, 0))
    return pl.pallas_call(
        _epi_body,
        grid=(N_G // R,),
        in_specs=[
            mat(),
            pl.BlockSpec((R, 1), lambda i: (i, 0)),
            mat(),
            mat(),
            wgt(), vec(), wgt(), vec(), wgt(), vec(),
        ],
        out_specs=mat(),
        out_shape=jax.ShapeDtypeStruct((N_G, D), jnp.float32),
    )(agg, cnt, x_g, eps, W_r, b_l.reshape(1, D), W_mu, b_mu.reshape(1, D),
      W_ls, b_ls.reshape(1, D))


def kernel(x_disease, x_gene, src_disease, dst_gene,
           W_l_dg, b_l_dg, W_r_dg, W_l_gd, b_l_gd, W_r_gd,
           W_mu, b_mu, W_ls, b_ls):
    y = _pre_matmul(x_disease, W_l_dg)
    agg = jax.ops.segment_sum(jnp.take(y, src_disease, axis=0), dst_gene,
                              num_segments=N_G)
    cnt = jax.ops.segment_sum(jnp.ones((E,), jnp.float32), dst_gene,
                              num_segments=N_G)
    eps = jax.random.normal(jax.random.key(42), (N_G, D), jnp.float32)
    return _epilogue(agg, cnt.reshape(N_G, 1), x_gene, eps,
                     W_r_dg, b_l_dg, W_mu, b_mu, W_ls, b_ls)


# speedup vs baseline: 4.7141x; 4.3099x over previous
import jax
import jax.numpy as jnp
from jax import lax
from jax.experimental import pallas as pl
from jax.experimental.pallas import tpu as pltpu
from jax.experimental.pallas import tpu_sc as plsc

N_D, N_G, E, D = 10000, 50000, 320000, 128

NC, NS = 2, 16
NCHUNK = 4
CHUNK = 12544
RPT = CHUNK // NS
NG_PAD = NCHUNK * CHUNK
E_TILE = E // NS
S = 2000
NSEG = E_TILE // S
K = 128
CB = S + K
DUMP = CHUNK


def _mm_body(x_ref, w_ref, y_ref):
    y_ref[...] = jnp.dot(x_ref[...], w_ref[...],
                         preferred_element_type=jnp.float32)


def _pre_matmul(x_d, W_l):
    R = 1000
    return pl.pallas_call(
        _mm_body,
        grid=(N_D // R,),
        in_specs=[
            pl.BlockSpec((R, D), lambda i: (i, 0)),
            pl.BlockSpec((D, D), lambda i: (0, 0)),
        ],
        out_specs=pl.BlockSpec((R, D), lambda i: (i, 0)),
        out_shape=jax.ShapeDtypeStruct((N_D, D), jnp.float32),
    )(x_d, W_l)


def _sc_body(y_hbm, src_hbm, dst_hbm, out_agg, out_cnt,
             comp_src, comp_dst, srcbuf, dstbuf, rowbuf, idx_stage,
             ones_buf, zcnt, agg_spmem, cnt_spmem, sem):
    cid = lax.axis_index("c")
    sid = lax.axis_index("s")

    zeros16 = jnp.zeros((16,), jnp.float32)

    def _fill_ones(t, _):
        ones_buf[pl.ds(t * 16, 16)] = jnp.ones((16,), jnp.float32)
        return 0
    lax.fori_loop(0, K // 16, _fill_ones, 0)

    def _fill_zcnt(t, _):
        zcnt[pl.ds(t * 16, 16)] = zeros16
        return 0
    lax.fori_loop(0, RPT // 16, _fill_zcnt, 0)

    for cc in range(2):
        chunk = cid * 2 + cc
        lo = pl.multiple_of(chunk * CHUNK, CHUNK)

        def _zrow(r, _):
            def _zcol(t, _):
                rowbuf[r, pl.ds(t * 16, 16)] = zeros16
                return 0
            lax.fori_loop(0, D // 16, _zcol, 0)
            return 0
        lax.fori_loop(0, K, _zrow, 0)

        zbase = sid * RPT
        for j in range(RPT // K):
            pltpu.sync_copy(rowbuf, agg_spmem.at[pl.ds(zbase + j * K, K)])
        rem = RPT - (RPT // K) * K
        pltpu.sync_copy(rowbuf.at[pl.ds(0, rem)],
                        agg_spmem.at[pl.ds(zbase + RPT - rem, rem)])
        pltpu.sync_copy(zcnt, cnt_spmem.at[pl.ds(zbase, RPT)])

        plsc.subcore_barrier()

        pad_dst = jnp.full((16,), DUMP, jnp.int32) + sid
        pad_src = jnp.full((16,), 0, jnp.int32) + sid * 625

        for seg in range(NSEG):
            ebase = pl.multiple_of(sid * E_TILE + seg * S, S)
            pltpu.sync_copy(dst_hbm.at[pl.ds(ebase, S)], dstbuf)
            pltpu.sync_copy(src_hbm.at[pl.ds(ebase, S)], srcbuf)

            def _compact(i, off):
                dv = dstbuf[pl.ds(i * 16, 16)]
                sv = srcbuf[pl.ds(i * 16, 16)]
                m = (dv >= lo) & (dv < lo + CHUNK)
                prefix = plsc.cumsum(m.astype(jnp.int32))
                pos = off + prefix - 1
                plsc.store_scatter(comp_dst, [pos], dv - lo, mask=m)
                plsc.store_scatter(comp_src, [pos], sv, mask=m)
                return off + jnp.sum(m.astype(jnp.int32))
            off = lax.fori_loop(0, S // 16, _compact, 0)

            def _pad(t, _):
                comp_dst[pl.ds(off + t * 16, 16)] = pad_dst
                comp_src[pl.ds(off + t * 16, 16)] = pad_src
                return 0
            lax.fori_loop(0, K // 16, _pad, 0)

            nblk = (off + K - 1) // K

            def _block(j, _):
                base = pl.multiple_of(j * K, K)
                pltpu.async_copy(y_hbm.at[comp_src.at[pl.ds(base, K)]],
                                 rowbuf, sem).wait()

                def _stage(t, _):
                    idx_stage[pl.ds(t * 16, 16)] = (
                        comp_dst[pl.ds(base + t * 16, 16)])
                    return 0
                lax.fori_loop(0, K // 16, _stage, 0)

                pltpu.sync_copy(rowbuf, agg_spmem.at[idx_stage], add=True)
                pltpu.sync_copy(ones_buf, cnt_spmem.at[idx_stage], add=True)
                return 0
            lax.fori_loop(0, nblk, _block, 0)

        plsc.subcore_barrier()

        obase = pl.multiple_of(lo + sid * RPT, RPT)
        pltpu.sync_copy(agg_spmem.at[pl.ds(zbase, RPT)],
                        out_agg.at[pl.ds(obase, RPT)])
        pltpu.sync_copy(cnt_spmem.at[pl.ds(zbase, RPT)], zcnt)
        pltpu.sync_copy(zcnt, out_cnt.at[pl.ds(obase, RPT)])
        def _fill_zcnt2(t, _):
            zcnt[pl.ds(t * 16, 16)] = zeros16
            return 0
        lax.fori_loop(0, RPT // 16, _fill_zcnt2, 0)

        plsc.subcore_barrier()


def _sc_segment_sum(y, src, dst):
    return pl.kernel(
        _sc_body,
        out_type=(jax.ShapeDtypeStruct((NG_PAD, D), jnp.float32),
                  jax.ShapeDtypeStruct((NG_PAD,), jnp.float32)),
        mesh=plsc.VectorSubcoreMesh(core_axis_name="c", subcore_axis_name="s"),
        compiler_params=pltpu.CompilerParams(needs_layout_passes=False),
        scratch_types=[
            pltpu.VMEM((CB,), jnp.int32),
            pltpu.VMEM((CB,), jnp.int32),
            pltpu.VMEM((S,), jnp.int32),
            pltpu.VMEM((S,), jnp.int32),
            pltpu.VMEM((K, D), jnp.float32),
            pltpu.VMEM((K,), jnp.int32),
            pltpu.VMEM((K,), jnp.float32),
            pltpu.VMEM((RPT,), jnp.float32),
            pltpu.VMEM_SHARED((CHUNK + NS, D), jnp.float32),
            pltpu.VMEM_SHARED((CHUNK + NS,), jnp.float32),
            pltpu.SemaphoreType.DMA,
        ],
    )(y, src, dst)


def _epi_body(agg_ref, cnt_ref, xg_ref, eps_ref, wr_ref, bl_ref,
              wmu_ref, bmu_ref, wls_ref, bls_ref, z_ref):
    cnt = jnp.maximum(cnt_ref[...], 1.0)
    h = (agg_ref[...] / cnt + bl_ref[...]
         + jnp.dot(xg_ref[...], wr_ref[...],
                   preferred_element_type=jnp.float32))
    mu = jnp.dot(h, wmu_ref[...], preferred_element_type=jnp.float32) + bmu_ref[...]
    ls = jnp.dot(h, wls_ref[...], preferred_element_type=jnp.float32) + bls_ref[...]
    z_ref[...] = mu + eps_ref[...] * jnp.exp(ls)


def _epilogue(agg, cnt, x_g, eps, W_r, b_l, W_mu, b_mu, W_ls, b_ls):
    R = 1000
    mat = lambda: pl.BlockSpec((R, D), lambda i: (i, 0))
    wgt = lambda: pl.BlockSpec((D, D), lambda i: (0, 0))
    vec = lambda: pl.BlockSpec((1, D), lambda i: (0, 0))
    return pl.pallas_call(
        _epi_body,
        grid=(N_G // R,),
        in_specs=[
            mat(),
            pl.BlockSpec((R, 1), lambda i: (i, 0)),
            mat(),
            mat(),
            wgt(), vec(), wgt(), vec(), wgt(), vec(),
        ],
        out_specs=mat(),
        out_shape=jax.ShapeDtypeStruct((N_G, D), jnp.float32),
    )(agg, cnt, x_g, eps, W_r, b_l.reshape(1, D), W_mu, b_mu.reshape(1, D),
      W_ls, b_ls.reshape(1, D))


def kernel(x_disease, x_gene, src_disease, dst_gene,
           W_l_dg, b_l_dg, W_r_dg, W_l_gd, b_l_gd, W_r_gd,
           W_mu, b_mu, W_ls, b_ls):
    y = _pre_matmul(x_disease, W_l_dg)
    agg, cnt = _sc_segment_sum(y, src_disease, dst_gene)
    eps = jax.random.normal(jax.random.key(42), (N_G, D), jnp.float32)
    return _epilogue(agg, cnt.reshape(NG_PAD, 1), x_gene, eps,
                     W_r_dg, b_l_dg, W_mu, b_mu, W_ls, b_ls)


# speedup vs baseline: 5.2867x; 1.1215x over previous
import jax
import jax.numpy as jnp
from jax import lax
from jax.experimental import pallas as pl
from jax.experimental.pallas import tpu as pltpu
from jax.experimental.pallas import tpu_sc as plsc

N_D, N_G, E, D = 10000, 50000, 320000, 128

NC, NS = 2, 16
NCHUNK = 6
CHUNK = 8448
RPT = CHUNK // NS
NG_PAD = NCHUNK * CHUNK
E_TILE = E // NS
S = 4000
NSEG = E_TILE // S
K = 128
NBLK_MAX = (S + 2 * K - 1) // K
DUMP = CHUNK


def _mm_body(x_ref, w_ref, y_ref):
    y_ref[...] = jnp.dot(x_ref[...], w_ref[...],
                         preferred_element_type=jnp.float32)


def _pre_matmul(x_d, W_l):
    R = 1000
    return pl.pallas_call(
        _mm_body,
        grid=(N_D // R,),
        in_specs=[
            pl.BlockSpec((R, D), lambda i: (i, 0)),
            pl.BlockSpec((D, D), lambda i: (0, 0)),
        ],
        out_specs=pl.BlockSpec((R, D), lambda i: (i, 0)),
        out_shape=jax.ShapeDtypeStruct((N_D, D), jnp.float32),
    )(x_d, W_l)


def _sc_body(y_hbm, src_hbm, dst_hbm, zeros_hbm, out_agg, out_cnt,
             comp_src, comp_dst, srcbuf0, srcbuf1, dstbuf0, dstbuf1,
             rowbuf0, rowbuf1, ones_buf, zcnt, cntb,
             agg_spmem, cnt_spmem, semE0, semE1, gsem0, gsem1):
    cid = lax.axis_index("c")
    sid = lax.axis_index("s")
    iota16 = lax.iota(jnp.int32, 16)
    zeros16 = jnp.zeros((16,), jnp.float32)

    def _fill_ones(t, _):
        ones_buf[pl.ds(t * 16, 16)] = jnp.ones((16,), jnp.float32)
        return 0
    lax.fori_loop(0, K // 16, _fill_ones, 0)

    def _fill_zcnt(t, _):
        zcnt[pl.ds(t * 16, 16)] = zeros16
        return 0
    lax.fori_loop(0, RPT // 16, _fill_zcnt, 0)

    pad_dst = jnp.full((16,), DUMP, jnp.int32) + sid
    pad_src = jnp.full((16,), 0, jnp.int32) + sid * 625
    ebufs = ((srcbuf0, dstbuf0, semE0), (srcbuf1, dstbuf1, semE1))

    for cc in range(NCHUNK // NC):
        chunk = cid * (NCHUNK // NC) + cc
        lo = chunk * CHUNK
        zbase = sid * RPT

        for q in range(RPT // K):
            pltpu.sync_copy(zeros_hbm,
                            agg_spmem.at[pl.ds(zbase + q * K, K)])
        rem = RPT - (RPT // K) * K
        pltpu.sync_copy(zeros_hbm.at[pl.ds(0, rem)],
                        agg_spmem.at[pl.ds(zbase + RPT - rem, rem)])
        pltpu.sync_copy(zcnt, cnt_spmem.at[pl.ds(zbase, RPT)])

        plsc.subcore_barrier()

        sb, db, se = ebufs[0]
        eb0 = pl.multiple_of(sid * E_TILE, S)
        pend = (pltpu.async_copy(dst_hbm.at[pl.ds(eb0, S)], db, se),
                pltpu.async_copy(src_hbm.at[pl.ds(eb0, S)], sb, se))

        for seg in range(NSEG):
            sb, db, se = ebufs[seg % 2]
            pend[0].wait()
            pend[1].wait()
            if seg + 1 < NSEG:
                nsb, ndb, nse = ebufs[(seg + 1) % 2]
                ebn = pl.multiple_of(sid * E_TILE + (seg + 1) * S, S)
                pend = (pltpu.async_copy(dst_hbm.at[pl.ds(ebn, S)], ndb, nse),
                        pltpu.async_copy(src_hbm.at[pl.ds(ebn, S)], nsb, nse))

            def _compact(i, off, db=db, sb=sb, lo=lo):
                dv = db[pl.ds(i * 16, 16)]
                sv = sb[pl.ds(i * 16, 16)]
                m = (dv >= lo) & (dv < lo + CHUNK)
                pr = plsc.cumsum(m.astype(jnp.int32))
                pos = off + pr - 1
                r = jnp.right_shift(pos, 7)
                c = jnp.bitwise_and(pos, 127)
                plsc.store_scatter(comp_dst, [r, c], dv - lo, mask=m)
                plsc.store_scatter(comp_src, [r, c], sv, mask=m)
                return off + pr[15]
            off = lax.fori_loop(0, S // 16, _compact, 0)

            def _pad(t, _, off=off):
                pos = off + t * 16 + iota16
                r = jnp.right_shift(pos, 7)
                c = jnp.bitwise_and(pos, 127)
                plsc.store_scatter(comp_dst, [r, c], pad_dst)
                plsc.store_scatter(comp_src, [r, c], pad_src)
                return 0
            lax.fori_loop(0, K // 16, _pad, 0)

            nblk = (off + K - 1) // K

            @pl.when(nblk > 0)
            def _():
                pltpu.async_copy(y_hbm.at[comp_src.at[0]], rowbuf0, gsem0)

            def _pair(p, _, nblk=nblk):
                j0 = p * 2
                j1 = j0 + 1

                @pl.when(j1 < nblk)
                def _():
                    pltpu.async_copy(y_hbm.at[comp_src.at[j1]],
                                     rowbuf1, gsem1)

                pltpu.make_async_copy(y_hbm.at[pl.ds(0, K)],
                                      rowbuf0, gsem0).wait()
                pltpu.sync_copy(rowbuf0, agg_spmem.at[comp_dst.at[j0]],
                                add=True)
                pltpu.sync_copy(ones_buf, cnt_spmem.at[comp_dst.at[j0]],
                                add=True)

                @pl.when(j0 + 2 < nblk)
                def _():
                    pltpu.async_copy(y_hbm.at[comp_src.at[j0 + 2]],
                                     rowbuf0, gsem0)

                @pl.when(j1 < nblk)
                def _():
                    pltpu.make_async_copy(y_hbm.at[pl.ds(0, K)],
                                          rowbuf1, gsem1).wait()
                    pltpu.sync_copy(rowbuf1, agg_spmem.at[comp_dst.at[j1]],
                                    add=True)
                    pltpu.sync_copy(ones_buf, cnt_spmem.at[comp_dst.at[j1]],
                                    add=True)
                return 0
            lax.fori_loop(0, (nblk + 1) // 2, _pair, 0)

        plsc.subcore_barrier()

        obase = pl.multiple_of(lo + sid * RPT, 16)
        pltpu.sync_copy(agg_spmem.at[pl.ds(zbase, RPT)],
                        out_agg.at[pl.ds(obase, RPT)])
        pltpu.sync_copy(cnt_spmem.at[pl.ds(zbase, RPT)], cntb)
        pltpu.sync_copy(cntb, out_cnt.at[pl.ds(obase, RPT)])

        plsc.subcore_barrier()


def _sc_segment_sum(y, src, dst, zeros128):
    return pl.kernel(
        _sc_body,
        out_type=(jax.ShapeDtypeStruct((NG_PAD, D), jnp.float32),
                  jax.ShapeDtypeStruct((NG_PAD,), jnp.float32)),
        mesh=plsc.VectorSubcoreMesh(core_axis_name="c", subcore_axis_name="s"),
        compiler_params=pltpu.CompilerParams(needs_layout_passes=False),
        scratch_types=[
            pltpu.VMEM((NBLK_MAX, K), jnp.int32),
            pltpu.VMEM((NBLK_MAX, K), jnp.int32),
            pltpu.VMEM((S,), jnp.int32),
            pltpu.VMEM((S,), jnp.int32),
            pltpu.VMEM((S,), jnp.int32),
            pltpu.VMEM((S,), jnp.int32),
            pltpu.VMEM((K, D), jnp.float32),
            pltpu.VMEM((K, D), jnp.float32),
            pltpu.VMEM((K,), jnp.float32),
            pltpu.VMEM((RPT,), jnp.float32),
            pltpu.VMEM((RPT,), jnp.float32),
            pltpu.VMEM_SHARED((CHUNK + NS, D), jnp.float32),
            pltpu.VMEM_SHARED((CHUNK + NS,), jnp.float32),
            pltpu.SemaphoreType.DMA,
            pltpu.SemaphoreType.DMA,
            pltpu.SemaphoreType.DMA,
            pltpu.SemaphoreType.DMA,
        ],
    )(y, src, dst, zeros128)


def _epi_body(agg_ref, cnt_ref, xg_ref, eps_ref, wr_ref, bl_ref,
              wmu_ref, bmu_ref, wls_ref, bls_ref, z_ref):
    cnt = jnp.maximum(cnt_ref[...], 1.0)
    h = (agg_ref[...] / cnt + bl_ref[...]
         + jnp.dot(xg_ref[...], wr_ref[...],
                   preferred_element_type=jnp.float32))
    mu = jnp.dot(h, wmu_ref[...], preferred_element_type=jnp.float32) + bmu_ref[...]
    ls = jnp.dot(h, wls_ref[...], preferred_element_type=jnp.float32) + bls_ref[...]
    z_ref[...] = mu + eps_ref[...] * jnp.exp(ls)


def _epilogue(agg, cnt, x_g, eps, W_r, b_l, W_mu, b_mu, W_ls, b_ls):
    R = 1000
    mat = lambda: pl.BlockSpec((R, D), lambda i: (i, 0))
    wgt = lambda: pl.BlockSpec((D, D), lambda i: (0, 0))
    vec = lambda: pl.BlockSpec((1, D), lambda i: (0, 0))
    return pl.pallas_call(
        _epi_body,
        grid=(N_G // R,),
        in_specs=[
            mat(),
            pl.BlockSpec((R, 1), lambda i: (i, 0)),
            mat(),
            mat(),
            wgt(), vec(), wgt(), vec(), wgt(), vec(),
        ],
        out_specs=mat(),
        out_shape=jax.ShapeDtypeStruct((N_G, D), jnp.float32),
    )(agg, cnt, x_g, eps, W_r, b_l.reshape(1, D), W_mu, b_mu.reshape(1, D),
      W_ls, b_ls.reshape(1, D))


def kernel(x_disease, x_gene, src_disease, dst_gene,
           W_l_dg, b_l_dg, W_r_dg, W_l_gd, b_l_gd, W_r_gd,
           W_mu, b_mu, W_ls, b_ls):
    y = _pre_matmul(x_disease, W_l_dg)
    zeros128 = jnp.zeros((K, D), jnp.float32)
    agg, cnt = _sc_segment_sum(y, src_disease, dst_gene, zeros128)
    eps = jax.random.normal(jax.random.key(42), (N_G, D), jnp.float32)
    return _epilogue(agg, cnt.reshape(NG_PAD, 1), x_gene, eps,
                     W_r_dg, b_l_dg, W_mu, b_mu, W_ls, b_ls)
